# Initial kernel scaffold; baseline (speedup 1.0000x reference)
import jax
import jax.numpy as jnp
from jax.experimental import pallas as pl


def kernel(x, a, i, W1, b1, W2, b2, g1, be1, g2, be2, Wf, bf, Wa, ba, Wo, bo):
    raise NotImplementedError("write your pallas kernel here")


# speedup vs baseline: 2.0492x; 2.0492x over previous
import jax
import jax.numpy as jnp
from jax import lax
from jax.experimental import pallas as pl
from jax.experimental.pallas import tpu as pltpu

N = 10000
F = 128
H = 64
B = 64
OUT = 3
EPS = 1e-3
R = 200
NR = N // R


def _ln_rows(h, gamma, beta):
    mu = jnp.mean(h, axis=-1, keepdims=True)
    var = jnp.mean((h - mu) ** 2, axis=-1, keepdims=True)
    return (h - mu) / jnp.sqrt(var + EPS) * gamma + beta


def _gnn_body(a_ref, xf_ref, xb_ref, seg_ref, W1_ref, b1_ref, W2_ref, b2_ref,
              g1_ref, be1_ref, g2_ref, be2_ref, Wf_ref, bf_ref, Wa_ref,
              ba_ref, Wo_ref, bo_ref, out_ref,
              u1_scr, u2_scr, h1_scr, g_acc, v_acc):
    p = pl.program_id(0)
    r = pl.program_id(1)

    @pl.when(jnp.logical_and(p == 0, r == 0))
    def _():
        u1_scr[...] = jnp.dot(xf_ref[...], W1_ref[...],
                              preferred_element_type=jnp.float32)

    @pl.when(p == 0)
    def _():
        acc = jnp.dot(a_ref[...], u1_scr[...],
                      preferred_element_type=jnp.float32)
        h = jax.nn.relu(acc + b1_ref[...])
        h = _ln_rows(h, g1_ref[...], be1_ref[...])
        h1_scr[pl.ds(r * R, R), :] = h
        u2_scr[pl.ds(r * R, R), :] = jnp.dot(
            h, W2_ref[...], preferred_element_type=jnp.float32)

    @pl.when(p == 1)
    def _():
        acc = jnp.dot(a_ref[...], u2_scr[...],
                      preferred_element_type=jnp.float32)
        h = jax.nn.relu(acc + b2_ref[...])
        h = _ln_rows(h, g2_ref[...], be2_ref[...])
        h = h + h1_scr[pl.ds(r * R, R), :]
        feat = jnp.dot(h, Wf_ref[...],
                       preferred_element_type=jnp.float32) + bf_ref[...]
        attn = jax.nn.sigmoid(
            jnp.dot(h, Wa_ref[...], preferred_element_type=jnp.float32)
            + ba_ref[...])
        gated = feat * attn

        xb = xb_ref[...]
        pcol = 1.0 + jax.nn.relu(xb[:, 0:1])
        vals = jnp.concatenate([pcol * xb[:, F - 3:F], pcol], axis=1)

        onehot = (seg_ref[...] == lax.broadcasted_iota(
            jnp.int32, (R, B), 1)).astype(jnp.float32)
        gseg = lax.dot_general(onehot, gated, (((0,), (0,)), ((), ())),
                               preferred_element_type=jnp.float32)
        vseg = lax.dot_general(onehot, vals, (((0,), (0,)), ((), ())),
                               preferred_element_type=jnp.float32)

        @pl.when(r == 0)
        def _():
            g_acc[...] = gseg
            v_acc[...] = vseg

        @pl.when(r > 0)
        def _():
            g_acc[...] += gseg
            v_acc[...] += vseg

        @pl.when(r == NR - 1)
        def _():
            g = g_acc[...]
            v = v_acc[...]
            bary = v[:, 0:3] / jnp.maximum(v[:, 3:4], 1e-30)
            out = (jnp.dot(g, Wo_ref[0:H, :],
                           preferred_element_type=jnp.float32)
                   + jnp.dot(bary, Wo_ref[H:H + 3, :],
                             preferred_element_type=jnp.float32)
                   + bo_ref[...])
            out_ref[...] = out


def kernel(x, a, i, W1, b1, W2, b2, g1, be1, g2, be2, Wf, bf, Wa, ba, Wo, bo):
    seg = i.astype(jnp.int32).reshape(N, 1)
    b1r = b1.reshape(1, H)
    b2r = b2.reshape(1, H)
    g1r = g1.reshape(1, H)
    be1r = be1.reshape(1, H)
    g2r = g2.reshape(1, H)
    be2r = be2.reshape(1, H)
    bfr = bf.reshape(1, H)
    bar = ba.reshape(1, H)
    bor = bo.reshape(1, OUT)

    full = lambda shape: pl.BlockSpec(shape, lambda p, r: (0, 0))
    rows = lambda shape: pl.BlockSpec(shape, lambda p, r: (r, 0))

    out = pl.pallas_call(
        _gnn_body,
        grid=(2, NR),
        in_specs=[
            rows((R, N)),
            full((N, F)),
            rows((R, F)),
            rows((R, 1)),
            full((F, H)),
            full((1, H)),
            full((H, H)),
            full((1, H)),
            full((1, H)),
            full((1, H)),
            full((1, H)),
            full((1, H)),
            full((H, H)),
            full((1, H)),
            full((H, H)),
            full((1, H)),
            full((H + 3, OUT)),
            full((1, OUT)),
        ],
        out_specs=pl.BlockSpec((B, OUT), lambda p, r: (0, 0)),
        out_shape=jax.ShapeDtypeStruct((B, OUT), jnp.float32),
        scratch_shapes=[
            pltpu.VMEM((N, H), jnp.float32),
            pltpu.VMEM((N, H), jnp.float32),
            pltpu.VMEM((N, H), jnp.float32),
            pltpu.VMEM((B, H), jnp.float32),
            pltpu.VMEM((B, 4), jnp.float32),
        ],
    )(a, x, x, seg, W1, b1r, W2, b2r, g1r, be1r, g2r, be2r,
      Wf, bfr, Wa, bar, Wo, bor)
    return out
